# Initial kernel scaffold; baseline (speedup 1.0000x reference)
import jax
import jax.numpy as jnp
from jax.experimental import pallas as pl


def kernel(adj_indices, adj_values, embedding_table):
    raise NotImplementedError("write your pallas kernel here")


# speedup vs baseline: 1.5053x; 1.5053x over previous
import functools

import jax
import jax.numpy as jnp
from jax import lax
from jax.experimental import pallas as pl
from jax.experimental.pallas import tpu as pltpu
from jax.experimental.pallas import tpu_sc as plsc

_N = 40727
_EMB = 100
_D = 128
_CH = 6848
_NPAD = 6 * _CH
_SROWS = _CH + 16
_ZROWS = _SROWS // 16
_NNZ = 651632
_EPW = 40960
_K = 512
_NCHUNK = _EPW // _K
_NNZ_PAD = 16 * _EPW


def _sc_layer(rows, cols, vals, cur, zeros):
    mesh = plsc.VectorSubcoreMesh(core_axis_name="c", subcore_axis_name="s")

    @functools.partial(
        pl.kernel,
        mesh=mesh,
        out_type=jax.ShapeDtypeStruct((_NPAD, _D), jnp.float32),
        scratch_types=[
            pltpu.VMEM((_K,), jnp.int32),
            pltpu.VMEM((_K,), jnp.int32),
            pltpu.VMEM((_K,), jnp.int32),
            pltpu.VMEM((_K,), jnp.float32),
            pltpu.VMEM((_K, _D), jnp.float32),
            pltpu.VMEM_SHARED((_SROWS, _D), jnp.float32),
            pltpu.SemaphoreType.DMA,
        ],
    )
    def k(rows_hbm, cols_hbm, vals_hbm, cur_hbm, zeros_hbm, out_hbm,
          cols_t, rows_t, lidx_t, vals_t, gbuf, shared, sem):
        c = lax.axis_index("c")
        s = lax.axis_index("s")
        base = s * _EPW

        for p in range(3):
            row0 = (3 * c + p) * _CH

            pltpu.sync_copy(zeros_hbm, shared.at[pl.ds(s * _ZROWS, _ZROWS)])
            plsc.subcore_barrier()

            @pl.loop(0, _NCHUNK)
            def _chunk(kk):
                off = base + kk * _K
                pltpu.sync_copy(cols_hbm.at[pl.ds(off, _K)], cols_t)
                pltpu.sync_copy(rows_hbm.at[pl.ds(off, _K)], rows_t)
                pltpu.sync_copy(vals_hbm.at[pl.ds(off, _K)], vals_t)
                pltpu.async_copy(cur_hbm.at[cols_t], gbuf, sem).wait()

                @pl.loop(0, _K // 16)
                def _grp(g):
                    e0 = g * 16
                    lv = rows_t[pl.ds(e0, 16)] - row0
                    oob = (lv < 0) | (lv >= _CH)
                    lidx_t[pl.ds(e0, 16)] = jnp.where(oob, _CH, lv)

                @pl.loop(0, _K // 16)
                def _edge16(g):
                    e0 = g * 16
                    vblk = vals_t[pl.ds(e0, 16)]
                    for l in range(16):
                        vv = vblk.at[jnp.full((16,), l, jnp.int32)].get(
                            mode="promise_in_bounds")
                        e = e0 + l
                        for d0 in range(0, _D, 16):
                            gbuf[e, pl.ds(d0, 16)] = (
                                gbuf[e, pl.ds(d0, 16)] * vv)

                pltpu.sync_copy(gbuf, shared.at[lidx_t], add=True)

            plsc.subcore_barrier()

            @pl.when(s == 0)
            def _():
                pltpu.sync_copy(shared.at[pl.ds(0, _CH)],
                                out_hbm.at[pl.ds(row0, _CH)])

            plsc.subcore_barrier()

    return k(rows, cols, vals, cur, zeros)


_BLK = 512


def _mean_body(e_ref, a_ref, b_ref, c_ref, o_ref):
    o_ref[...] = (e_ref[...] + a_ref[...] + b_ref[...] + c_ref[...]) * 0.25


def _mean4(emb, o1, o2, o3):
    spec = pl.BlockSpec((_BLK, _D), lambda i: (i, 0))
    return pl.pallas_call(
        _mean_body,
        grid=(pl.cdiv(_NPAD, _BLK),),
        in_specs=[spec, spec, spec, spec],
        out_specs=spec,
        out_shape=jax.ShapeDtypeStruct((_NPAD, _D), jnp.float32),
    )(emb, o1, o2, o3)


def kernel(adj_indices, adj_values, embedding_table):
    rows = adj_indices[0].astype(jnp.int32)
    cols = adj_indices[1].astype(jnp.int32)
    pad = _NNZ_PAD - rows.shape[0]
    rows = jnp.pad(rows, (0, pad))
    cols = jnp.pad(cols, (0, pad))
    vals = jnp.pad(adj_values.astype(jnp.float32), (0, pad))
    emb = jnp.pad(embedding_table.astype(jnp.float32),
                  ((0, _NPAD - _N), (0, _D - _EMB)))
    zeros = jnp.zeros((_ZROWS, _D), jnp.float32)

    cur = emb
    outs = []
    for _ in range(3):
        cur = _sc_layer(rows, cols, vals, cur, zeros)
        outs.append(cur)
    acc = _mean4(emb, outs[0], outs[1], outs[2])
    return acc[:_N, :_EMB]
